# baseline (speedup 1.0000x reference)
import functools
import math

import jax
import jax.numpy as jnp
from jax import lax
from jax.experimental import pallas as pl
from jax.experimental.pallas import tpu as pltpu
from jax.experimental.pallas import tpu_sc as plsc

_B = 32
_SEQ = 2500
_D = 1024
_LANES = 16
_CHUNK = 128
_SUBROWS = _D // _CHUNK
_SCALE = math.sqrt(_D)

_NC, _NS = 2, 16
_NW = _NC * _NS
_PER_W = (_B * _SUBROWS) // _NW


def _sc_body(idx_hbm, x_hbm, pe_hbm, alpha_hbm, out_hbm,
             idx_v, rows_v, x_v, alpha_v, sem):
    wid = lax.axis_index("s") * _NC + lax.axis_index("c")
    base = wid * _PER_W
    pltpu.sync_copy(idx_hbm.at[pl.ds(base, _PER_W)], idx_v)
    gather = pltpu.async_copy(pe_hbm.at[idx_v], rows_v, sem)
    pltpu.sync_copy(x_hbm.at[pl.ds(base, _PER_W)], x_v)
    pltpu.sync_copy(alpha_hbm, alpha_v)
    gather.wait()
    alpha_vec = alpha_v[...]
    for i in range(_PER_W):
        for j in range(_CHUNK // _LANES):
            sl = pl.ds(j * _LANES, _LANES)
            x_v[i, sl] = x_v[i, sl] * _SCALE + alpha_vec * rows_v[i, sl]
    pltpu.sync_copy(x_v, out_hbm.at[pl.ds(base, _PER_W)])


_sc_call = functools.partial(
    pl.kernel,
    mesh=plsc.VectorSubcoreMesh(core_axis_name="c", subcore_axis_name="s"),
    out_type=jax.ShapeDtypeStruct((_B * _SUBROWS, _CHUNK), jnp.float32),
    scratch_types=[
        pltpu.VMEM((_PER_W,), jnp.int32),
        pltpu.VMEM((_PER_W, _CHUNK), jnp.float32),
        pltpu.VMEM((_PER_W, _CHUNK), jnp.float32),
        pltpu.VMEM((_LANES,), jnp.float32),
        pltpu.SemaphoreType.DMA,
    ],
)(_sc_body)


@jax.jit
def kernel(input_pos, x, pe, alpha):
    pe_flat = pe.reshape(_B * _SEQ * _SUBROWS, _CHUNK)
    row_base = (jnp.arange(_B, dtype=jnp.int32) * _SEQ + (input_pos - 1)) * _SUBROWS
    idx = (row_base[:, None]
           + jnp.arange(_SUBROWS, dtype=jnp.int32)[None, :]).reshape(-1)
    x_flat = x.reshape(_B * _SUBROWS, _CHUNK)
    alpha16 = jnp.broadcast_to(alpha.astype(jnp.float32), (_LANES,))
    out = _sc_call(idx, x_flat, pe_flat, alpha16)
    return out.reshape(_B, 1, _D)


# speedup vs baseline: 1.9241x; 1.9241x over previous
import functools
import math

import jax
import jax.numpy as jnp
from jax import lax
from jax.experimental import pallas as pl
from jax.experimental.pallas import tpu as pltpu
from jax.experimental.pallas import tpu_sc as plsc

_B = 32
_SEQ = 2500
_D = 1024
_LANES = 16
_SCALE = math.sqrt(_D)

_NC, _NS = 2, 16
_NW = _NC * _NS
_IDX_PAD = 8


def _sc_body(idx_hbm, x_hbm, pe_hbm, alpha_hbm, out_hbm,
             idx_v, row_v, x_v, alpha_v, sem):
    wid = lax.axis_index("s") * _NC + lax.axis_index("c")
    pltpu.sync_copy(idx_hbm.at[pl.ds(wid * _IDX_PAD, _IDX_PAD)], idx_v)
    gather = pltpu.async_copy(
        pe_hbm.at[wid].at[idx_v.at[pl.ds(0, 1)]], row_v, sem)
    pltpu.sync_copy(x_hbm.at[wid], x_v)
    pltpu.sync_copy(alpha_hbm, alpha_v)
    gather.wait()
    alpha_vec = alpha_v[...]
    for j in range(_D // _LANES):
        sl = pl.ds(j * _LANES, _LANES)
        x_v[0, sl] = x_v[0, sl] * _SCALE + alpha_vec * row_v[0, sl]
    pltpu.sync_copy(x_v, out_hbm.at[wid])


_sc_call = functools.partial(
    pl.kernel,
    mesh=plsc.VectorSubcoreMesh(core_axis_name="c", subcore_axis_name="s"),
    out_type=jax.ShapeDtypeStruct((_B, 1, _D), jnp.float32),
    scratch_types=[
        pltpu.VMEM((_IDX_PAD,), jnp.int32),
        pltpu.VMEM((1, _D), jnp.float32),
        pltpu.VMEM((1, _D), jnp.float32),
        pltpu.VMEM((_LANES,), jnp.float32),
        pltpu.SemaphoreType.DMA,
    ],
)(_sc_body)


@jax.jit
def kernel(input_pos, x, pe, alpha):
    idx = jnp.broadcast_to((input_pos - 1)[:, None], (_B, _IDX_PAD)).reshape(-1)
    alpha16 = jnp.broadcast_to(alpha.astype(jnp.float32), (_LANES,))
    return _sc_call(idx, x, pe, alpha16)


# speedup vs baseline: 24.9871x; 12.9864x over previous
import functools
import math

import jax
import jax.numpy as jnp
from jax import lax
from jax.experimental import pallas as pl
from jax.experimental.pallas import tpu as pltpu
from jax.experimental.pallas import tpu_sc as plsc

_B = 32
_SEQ = 2500
_D = 1024
_LANES = 16
_SCALE = math.sqrt(_D)

_NC, _NS = 2, 16
_NW = _NC * _NS
_IDX_PAD = 8


def _sc_body(idx_hbm, x_hbm, pe_hbm, alpha_hbm, out_hbm,
             idx_v, row_v, x_v, alpha_v, sem):
    wid = lax.axis_index("s") * _NC + lax.axis_index("c")
    pltpu.sync_copy(idx_hbm.at[pl.ds(wid * _IDX_PAD, _IDX_PAD)], idx_v)
    gather = pltpu.async_copy(pe_hbm.at[idx_v.at[pl.ds(0, 1)]], row_v, sem)
    pltpu.sync_copy(x_hbm.at[wid], x_v)
    pltpu.sync_copy(alpha_hbm, alpha_v)
    gather.wait()
    alpha_vec = alpha_v[...]
    for j in range(_D // _LANES):
        sl = pl.ds(j * _LANES, _LANES)
        x_v[0, sl] = x_v[0, sl] * _SCALE + alpha_vec * row_v[0, sl]
    pltpu.sync_copy(x_v, out_hbm.at[wid])


_sc_call = functools.partial(
    pl.kernel,
    mesh=plsc.VectorSubcoreMesh(core_axis_name="c", subcore_axis_name="s"),
    out_type=jax.ShapeDtypeStruct((_B, 1, _D), jnp.float32),
    scratch_types=[
        pltpu.VMEM((_IDX_PAD,), jnp.int32),
        pltpu.VMEM((1, _D), jnp.float32),
        pltpu.VMEM((1, _D), jnp.float32),
        pltpu.VMEM((_LANES,), jnp.float32),
        pltpu.SemaphoreType.DMA,
    ],
)(_sc_body)


@jax.jit
def kernel(input_pos, x, pe, alpha):
    pe_rows = pe.transpose(1, 0, 2).reshape(_SEQ * _B, _D)
    flat = (input_pos - 1) * _B + jnp.arange(_B, dtype=jnp.int32)
    idx = jnp.broadcast_to(flat[:, None], (_B, _IDX_PAD)).reshape(-1)
    alpha16 = jnp.broadcast_to(alpha.astype(jnp.float32), (_LANES,))
    return _sc_call(idx, x, pe_rows, alpha16)


# speedup vs baseline: 25.3064x; 1.0128x over previous
import functools
import math

import jax
import jax.numpy as jnp
from jax import lax
from jax.experimental import pallas as pl
from jax.experimental.pallas import tpu as pltpu
from jax.experimental.pallas import tpu_sc as plsc

_B = 32
_SEQ = 2500
_D = 1024
_LANES = 16
_SCALE = math.sqrt(_D)

_NC, _NS = 2, 16
_NW = _NC * _NS


def _sc_body(pos_hbm, x_hbm, pe_hbm, alpha_hbm, out_hbm,
             pos_v, idx_v, row_v, x_v, alpha_v, sem, sem2):
    wid = lax.axis_index("s") * _NC + lax.axis_index("c")
    xcp = pltpu.async_copy(x_hbm.at[wid], x_v, sem2)
    acp = pltpu.async_copy(alpha_hbm, alpha_v, sem2)
    pltpu.sync_copy(pos_hbm, pos_v)
    half = wid // _LANES
    lane = wid % _LANES
    posv = pos_v[pl.ds(pl.multiple_of(half * _LANES, _LANES), _LANES)]
    batchv = half * _LANES + jnp.arange(_LANES, dtype=jnp.int32)
    flatv = (posv - 1) * _B + batchv
    plsc.store_scatter(idx_v, [jnp.arange(_LANES, dtype=jnp.int32) * 8], flatv)
    gather = pltpu.async_copy(
        pe_hbm.at[idx_v.at[pl.ds(pl.multiple_of(lane * 8, 8), 1)]], row_v, sem)
    acp.wait()
    xcp.wait()
    gather.wait()
    alpha_vec = alpha_v[...]
    for j in range(_D // _LANES):
        sl = pl.ds(j * _LANES, _LANES)
        x_v[0, sl] = x_v[0, sl] * _SCALE + alpha_vec * row_v[0, sl]
    pltpu.sync_copy(x_v, out_hbm.at[wid])


_sc_call = functools.partial(
    pl.kernel,
    mesh=plsc.VectorSubcoreMesh(core_axis_name="c", subcore_axis_name="s"),
    compiler_params=pltpu.CompilerParams(needs_layout_passes=False),
    out_type=jax.ShapeDtypeStruct((_B, 1, _D), jnp.float32),
    scratch_types=[
        pltpu.VMEM((_B,), jnp.int32),
        pltpu.VMEM((_LANES * 8,), jnp.int32),
        pltpu.VMEM((1, _D), jnp.float32),
        pltpu.VMEM((1, _D), jnp.float32),
        pltpu.VMEM((_LANES,), jnp.float32),
        pltpu.SemaphoreType.DMA,
        pltpu.SemaphoreType.DMA,
    ],
)(_sc_body)


@jax.jit
def kernel(input_pos, x, pe, alpha):
    pe_rows = pe.transpose(1, 0, 2).reshape(_SEQ * _B, _D)
    alpha16 = jnp.broadcast_to(alpha.astype(jnp.float32), (_LANES,))
    return _sc_call(input_pos, x, pe_rows, alpha16)


# speedup vs baseline: 25.6593x; 1.0139x over previous
import functools
import math

import jax
import jax.numpy as jnp
from jax import lax
from jax.experimental import pallas as pl
from jax.experimental.pallas import tpu as pltpu
from jax.experimental.pallas import tpu_sc as plsc

_B = 32
_SEQ = 2500
_D = 1024
_LANES = 16
_SCALE = math.sqrt(_D)

_NC, _NS = 2, 16
_NW = _NC * _NS


def _sc_body(pos_hbm, x_hbm, pe_hbm, alpha_hbm, out_hbm,
             pos_v, idx_v, row_v, x_v, alpha_v, sem, sem2):
    wid = lax.axis_index("s") * _NC + lax.axis_index("c")
    xcp = pltpu.async_copy(x_hbm.at[wid], x_v, sem2)
    acp = pltpu.async_copy(alpha_hbm, alpha_v, sem2)
    pltpu.sync_copy(pos_hbm, pos_v)
    half = wid // _LANES
    lane = wid % _LANES
    posv = pos_v[pl.ds(pl.multiple_of(half * _LANES, _LANES), _LANES)]
    batchv = half * _LANES + jnp.arange(_LANES, dtype=jnp.int32)
    flatv = (posv - 1) * _B + batchv
    plsc.store_scatter(idx_v, [jnp.arange(_LANES, dtype=jnp.int32) * 8], flatv)
    gather = pltpu.async_copy(
        pe_hbm.at[idx_v.at[pl.ds(pl.multiple_of(lane * 8, 8), 1)]], row_v, sem)
    acp.wait()
    xcp.wait()
    gather.wait()
    alpha_vec = plsc.load_gather(alpha_v, [jnp.zeros((_LANES,), jnp.int32)])

    def fma(j, carry):
        sl = pl.ds(pl.multiple_of(j * _LANES, _LANES), _LANES)
        x_v[0, sl] = x_v[0, sl] * _SCALE + alpha_vec * row_v[0, sl]
        return carry

    lax.fori_loop(0, _D // _LANES, fma, 0)
    pltpu.sync_copy(x_v, out_hbm.at[wid])


_sc_call = functools.partial(
    pl.kernel,
    mesh=plsc.VectorSubcoreMesh(core_axis_name="c", subcore_axis_name="s"),
    compiler_params=pltpu.CompilerParams(needs_layout_passes=False),
    out_type=jax.ShapeDtypeStruct((_B, 1, _D), jnp.float32),
    scratch_types=[
        pltpu.VMEM((_B,), jnp.int32),
        pltpu.VMEM((_LANES * 8,), jnp.int32),
        pltpu.VMEM((1, _D), jnp.float32),
        pltpu.VMEM((1, _D), jnp.float32),
        pltpu.VMEM((1,), jnp.float32),
        pltpu.SemaphoreType.DMA,
        pltpu.SemaphoreType.DMA,
    ],
)(_sc_body)


@jax.jit
def kernel(input_pos, x, pe, alpha):
    pe_rows = pe.transpose(1, 0, 2).reshape(_SEQ * _B, _D)
    return _sc_call(input_pos, x, pe_rows, alpha.astype(jnp.float32))
